# baseline (speedup 1.0000x reference)
import functools

import jax
import jax.numpy as jnp
from jax import lax
from jax.experimental import pallas as pl
from jax.experimental.pallas import tpu as pltpu
from jax.experimental.pallas import tpu_sc as plsc

N = 10000
E = 320000
F = 128
H = 256
NP = 10240
NC, NS = 2, 16
NW = NC * NS
RPP = 40
PPW = 2
EP = NW * PPW * RPP * 128
ROWS_PER_TILE = NP // NS


def _k1_body(xpad, srcp, dstp, zeros2d, out, sidx, didx, rows, agg_sh, sem):
    cid = lax.axis_index("c")
    sid = lax.axis_index("s")
    w = cid * NS + sid
    pltpu.sync_copy(zeros2d.at[pl.ds(sid * ROWS_PER_TILE, ROWS_PER_TILE)],
                    agg_sh.at[pl.ds(sid * ROWS_PER_TILE, ROWS_PER_TILE)])
    plsc.subcore_barrier()

    NB = 2
    for p in range(PPW):
        pltpu.sync_copy(srcp.at[w * PPW + p], sidx)
        pltpu.sync_copy(dstp.at[w * PPW + p], didx)
        for b in range(NB):
            pltpu.async_copy(xpad.at[sidx.at[b]], rows.at[b], sem.at[b])

        def body(t, carry):
            for b in range(NB):
                j = t * NB + b
                pltpu.make_async_copy(xpad.at[sidx.at[j]], rows.at[b],
                                      sem.at[b]).wait()
                pltpu.sync_copy(rows.at[b], agg_sh.at[didx.at[j]], add=True)

                @pl.when(j + NB < RPP)
                def _():
                    pltpu.async_copy(xpad.at[sidx.at[j + NB]], rows.at[b],
                                     sem.at[b])
            return carry

        lax.fori_loop(0, RPP // NB, body, 0)
    plsc.subcore_barrier()

    def body2(t, carry):
        r0 = sid * ROWS_PER_TILE + t * 128
        pltpu.sync_copy(agg_sh.at[pl.ds(r0, 128)], rows.at[0])
        pltpu.sync_copy(rows.at[0], out.at[pl.ds(cid * NP + r0, 128)])
        return carry

    lax.fori_loop(0, ROWS_PER_TILE // 128, body2, 0)


_k1 = functools.partial(
    pl.kernel,
    out_type=jax.ShapeDtypeStruct((NC * NP, F), jnp.float32),
    mesh=plsc.VectorSubcoreMesh(core_axis_name="c", subcore_axis_name="s",
                                num_cores=NC, num_subcores=NS),
    scratch_types=[
        pltpu.VMEM((RPP, 128), jnp.int32),
        pltpu.VMEM((RPP, 128), jnp.int32),
        pltpu.VMEM((2, 128, F), jnp.float32),
        pltpu.VMEM_SHARED((NP, F), jnp.float32),
        pltpu.SemaphoreType.DMA((2,)),
    ],
)(_k1_body)


def _k2_body(agg0, agg1, xb, wr, wo, b1r, wpr, wpo, bp1s, h_out, a_out, b_out):
    aggb = agg0[...] + agg1[...]
    h = jnp.dot(aggb, wr[...], preferred_element_type=jnp.float32)
    h += jnp.dot(xb[...], wo[...], preferred_element_type=jnp.float32)
    h = jnp.maximum(h + b1r[...], 0.0)
    h_out[...] = h
    a_out[...] = jnp.sum(h * wpr[...], axis=1).reshape(1, 1, -1)
    b_out[...] = (jnp.sum(h * wpo[...], axis=1) + bp1s[0, 0]).reshape(1, 1, -1)


def _k2(aggp, xpad, Wr1, Wo1, b1r, wpr1, wpo1, bp1s):
    R = 1024
    G = NP // R
    return pl.pallas_call(
        _k2_body,
        grid=(G,),
        in_specs=[
            pl.BlockSpec((R, F), lambda i: (i, 0)),
            pl.BlockSpec((R, F), lambda i: (i + G, 0)),
            pl.BlockSpec((R, F), lambda i: (i, 0)),
            pl.BlockSpec((F, H), lambda i: (0, 0)),
            pl.BlockSpec((F, H), lambda i: (0, 0)),
            pl.BlockSpec((1, H), lambda i: (0, 0)),
            pl.BlockSpec((1, H), lambda i: (0, 0)),
            pl.BlockSpec((1, H), lambda i: (0, 0)),
            pl.BlockSpec((1, 1), lambda i: (0, 0)),
        ],
        out_specs=[
            pl.BlockSpec((R, H), lambda i: (i, 0)),
            pl.BlockSpec((1, 1, R), lambda i: (i, 0, 0)),
            pl.BlockSpec((1, 1, R), lambda i: (i, 0, 0)),
        ],
        out_shape=[
            jax.ShapeDtypeStruct((NP, H), jnp.float32),
            jax.ShapeDtypeStruct((G, 1, R), jnp.float32),
            jax.ShapeDtypeStruct((G, 1, R), jnp.float32),
        ],
    )(aggp, aggp, xpad, Wr1, Wo1, b1r, wpr1, wpo1, bp1s)


def _k3_body(a_hbm, srcp, dstp, out, sidx, didx, vals, zb, score_sh, sem):
    cid = lax.axis_index("c")
    sid = lax.axis_index("s")
    w = cid * NS + sid

    @pl.when(sid == 0)
    def _zero():
        def zbody(i, carry):
            zb[pl.ds(i * 16, 16)] = jnp.zeros((16,), jnp.float32)
            return carry
        lax.fori_loop(0, NP // 16, zbody, 0)
        pltpu.sync_copy(zb, score_sh)

    pltpu.sync_copy(srcp.at[pl.ds(w * PPW, PPW)], sidx)
    pltpu.sync_copy(dstp.at[pl.ds(w * PPW, PPW)], didx)

    for p in range(PPW):
        def gfire(j, carry):
            pltpu.async_copy(a_hbm.at[sidx.at[p].at[j]], vals.at[p].at[j],
                             sem.at[0])
            return carry
        lax.fori_loop(0, RPP, gfire, 0)
    for p in range(PPW):
        def gdrain(j, carry):
            pltpu.make_async_copy(a_hbm.at[sidx.at[p].at[j]],
                                  vals.at[p].at[j], sem.at[0]).wait()
            return carry
        lax.fori_loop(0, RPP, gdrain, 0)
    plsc.subcore_barrier()

    for p in range(PPW):
        def sfire(j, carry):
            pltpu.async_copy(vals.at[p].at[j], score_sh.at[didx.at[p].at[j]],
                             sem.at[1], add=True)
            return carry
        lax.fori_loop(0, RPP, sfire, 0)
    for p in range(PPW):
        def sdrain(j, carry):
            pltpu.make_async_copy(vals.at[p].at[j],
                                  score_sh.at[didx.at[p].at[j]],
                                  sem.at[1]).wait()
            return carry
        lax.fori_loop(0, RPP, sdrain, 0)
    plsc.subcore_barrier()

    @pl.when(sid == 0)
    def _out():
        pltpu.sync_copy(score_sh, zb)
        pltpu.sync_copy(zb, out.at[cid])


_k3 = functools.partial(
    pl.kernel,
    out_type=jax.ShapeDtypeStruct((NC, NP), jnp.float32),
    mesh=plsc.VectorSubcoreMesh(core_axis_name="c", subcore_axis_name="s",
                                num_cores=NC, num_subcores=NS),
    scratch_types=[
        pltpu.VMEM((PPW, RPP, 128), jnp.int32),
        pltpu.VMEM((PPW, RPP, 128), jnp.int32),
        pltpu.VMEM((PPW, RPP, 128), jnp.float32),
        pltpu.VMEM((NP,), jnp.float32),
        pltpu.VMEM_SHARED((NP,), jnp.float32),
        pltpu.SemaphoreType.DMA((2,)),
    ],
)(_k3_body)


def _k4_body(scorep, bvec, h, edges,
             wr2, wo2, b2r, wpr2, wpo2, bp2s,
             wr3, wo3, b3r, wpr3, wpo3, bp3s,
             wmt, bmr, out):
    s = scorep[0:1, :] + scorep[1:2, :] + bvec[...]
    iota = lax.broadcasted_iota(jnp.int32, (1, NP), 1)
    s = jnp.where(iota < N, s, jnp.float32(-3.0e38))
    m = jnp.max(s)
    idx = jnp.min(jnp.where(s == m, iota, NP))
    xn = h[pl.ds(idx, 1), :] * jnp.tanh(m)
    e0 = edges[0]
    e1 = edges[1]
    cf = jnp.sum(jnp.where((e0 == idx) & (e1 == idx), 1.0, 0.0))

    def gconv(v, wr, wo, br):
        y = cf * jnp.dot(v, wr[...], preferred_element_type=jnp.float32)
        y += jnp.dot(v, wo[...], preferred_element_type=jnp.float32)
        return jnp.maximum(y + br[...], 0.0)

    g2 = gconv(xn, wr2, wo2, b2r)
    s2 = cf * jnp.sum(g2 * wpr2[...]) + jnp.sum(g2 * wpo2[...]) + bp2s[0, 0]
    xn2 = g2 * jnp.tanh(s2)
    g3 = gconv(xn2, wr3, wo3, b3r)
    s3 = cf * jnp.sum(g3 * wpr3[...]) + jnp.sum(g3 * wpo3[...]) + bp3s[0, 0]
    xn3 = g3 * jnp.tanh(s3)
    t = xn + xn2 + xn3
    o0 = jnp.sum(t * wmt[0:1, :]) + bmr[0, 0]
    o1 = jnp.sum(t * wmt[1:2, :]) + bmr[0, 1]
    out[...] = jnp.concatenate([o0.reshape(1, 1), o1.reshape(1, 1)], axis=1)


def _k4(scorep, bvec, h, edges, *ws):
    return pl.pallas_call(
        _k4_body,
        out_shape=jax.ShapeDtypeStruct((1, 2), jnp.float32),
    )(scorep, bvec, h, edges, *ws)


def kernel(x, edge_index, batch, Wr1, Wo1, b1, Wpr1, Wpo1, bp1,
           Wr2, Wo2, b2, Wpr2, Wpo2, bp2, Wr3, Wo3, b3, Wpr3, Wpo3, bp3,
           Wm, bm):
    src = edge_index[0]
    dst = edge_index[1]
    pad = jnp.full((EP - E,), N, jnp.int32)
    srcp = jnp.concatenate([src.astype(jnp.int32), pad]).reshape(
        NW * PPW, RPP, 128)
    dstp = jnp.concatenate([dst.astype(jnp.int32), pad]).reshape(
        NW * PPW, RPP, 128)
    xpad = jnp.concatenate([x, jnp.zeros((NP - N, F), jnp.float32)], axis=0)
    zeros2d = jnp.zeros((NP, F), jnp.float32)

    aggp = _k1(xpad, srcp, dstp, zeros2d)

    h, a3, b3v = _k2(aggp, xpad,
                     Wr1, Wo1, b1.reshape(1, H),
                     Wpr1.reshape(1, H), Wpo1.reshape(1, H),
                     bp1.reshape(1, 1))
    a1 = a3.reshape(NP)
    bvec = b3v.reshape(1, NP)

    scorep = _k3(a1, srcp, dstp)

    edges = edge_index.astype(jnp.int32).reshape(2, E // 128, 128)
    wmt = (Wm[:H] + Wm[H:]).T
    return _k4(scorep, bvec, h, edges,
               Wr2, Wo2, b2.reshape(1, H), Wpr2.reshape(1, H),
               Wpo2.reshape(1, H), bp2.reshape(1, 1),
               Wr3, Wo3, b3.reshape(1, H), Wpr3.reshape(1, H),
               Wpo3.reshape(1, H), bp3.reshape(1, 1),
               wmt, bm.reshape(1, 2))


# speedup vs baseline: 2.1438x; 2.1438x over previous
import functools

import jax
import jax.numpy as jnp
from jax import lax
from jax.experimental import pallas as pl
from jax.experimental.pallas import tpu as pltpu
from jax.experimental.pallas import tpu_sc as plsc

N = 10000
E = 320000
F = 128
H = 256
NP = 10240
NC, NS = 2, 16
NW = NC * NS
RPP = 40
PPW = 2
EP = NW * PPW * RPP * 128
ROWS_PER_TILE = NP // NS


FH = F // 2
TPP = 4


def _k1_body(xsplit, srcp, dstp, zeros2d, out, sidx, didx, rows,
             x_sh, agg_sh, sem):
    cid = lax.axis_index("c")
    sid = lax.axis_index("s")
    r_base = sid * ROWS_PER_TILE
    x_half = xsplit.at[pl.ds(cid * NP, NP)]
    pltpu.sync_copy(x_half.at[pl.ds(r_base, ROWS_PER_TILE)],
                    x_sh.at[pl.ds(r_base, ROWS_PER_TILE)])
    pltpu.sync_copy(zeros2d.at[pl.ds(r_base, ROWS_PER_TILE)],
                    agg_sh.at[pl.ds(r_base, ROWS_PER_TILE)])
    plsc.subcore_barrier()

    NB = 2
    for p in range(TPP):
        pltpu.sync_copy(srcp.at[sid * TPP + p], sidx)
        pltpu.sync_copy(dstp.at[sid * TPP + p], didx)
        for b in range(NB):
            pltpu.async_copy(x_sh.at[sidx.at[b]], rows.at[b], sem.at[b])

        def body(t, carry):
            for b in range(NB):
                j = t * NB + b
                pltpu.make_async_copy(x_sh.at[sidx.at[j]], rows.at[b],
                                      sem.at[b]).wait()
                pltpu.sync_copy(rows.at[b], agg_sh.at[didx.at[j]], add=True)

                @pl.when(j + NB < RPP)
                def _():
                    pltpu.async_copy(x_sh.at[sidx.at[j + NB]], rows.at[b],
                                     sem.at[b])
            return carry

        lax.fori_loop(0, RPP // NB, body, 0)
    plsc.subcore_barrier()

    def body2(t, carry):
        r0 = r_base + t * 128
        pltpu.sync_copy(agg_sh.at[pl.ds(r0, 128)], rows.at[0])
        pltpu.sync_copy(rows.at[0], out.at[pl.ds(cid * NP + r0, 128)])
        return carry

    lax.fori_loop(0, ROWS_PER_TILE // 128, body2, 0)


_k1 = functools.partial(
    pl.kernel,
    out_type=jax.ShapeDtypeStruct((NC * NP, FH), jnp.float32),
    mesh=plsc.VectorSubcoreMesh(core_axis_name="c", subcore_axis_name="s",
                                num_cores=NC, num_subcores=NS),
    scratch_types=[
        pltpu.VMEM((RPP, 128), jnp.int32),
        pltpu.VMEM((RPP, 128), jnp.int32),
        pltpu.VMEM((2, 128, FH), jnp.float32),
        pltpu.VMEM_SHARED((NP, FH), jnp.float32),
        pltpu.VMEM_SHARED((NP, FH), jnp.float32),
        pltpu.SemaphoreType.DMA((2,)),
    ],
    compiler_params=pltpu.CompilerParams(use_tc_tiling_on_sc=False),
)(_k1_body)


def _k2_body(agg0, agg1, xb, wr, wo, b1r, wpr, wpo, bp1s, h_out, a_out, b_out):
    aggb = jnp.concatenate([agg0[...], agg1[...]], axis=1)
    h = jnp.dot(aggb, wr[...], preferred_element_type=jnp.float32)
    h += jnp.dot(xb[...], wo[...], preferred_element_type=jnp.float32)
    h = jnp.maximum(h + b1r[...], 0.0)
    h_out[...] = h
    a_out[...] = jnp.sum(h * wpr[...], axis=1).reshape(1, 1, -1)
    b_out[...] = (jnp.sum(h * wpo[...], axis=1) + bp1s[0, 0]).reshape(1, 1, -1)


def _k2(aggp, xpad, Wr1, Wo1, b1r, wpr1, wpo1, bp1s):
    R = 1024
    G = NP // R
    return pl.pallas_call(
        _k2_body,
        grid=(G,),
        in_specs=[
            pl.BlockSpec((R, FH), lambda i: (i, 0)),
            pl.BlockSpec((R, FH), lambda i: (i + G, 0)),
            pl.BlockSpec((R, F), lambda i: (i, 0)),
            pl.BlockSpec((F, H), lambda i: (0, 0)),
            pl.BlockSpec((F, H), lambda i: (0, 0)),
            pl.BlockSpec((1, H), lambda i: (0, 0)),
            pl.BlockSpec((1, H), lambda i: (0, 0)),
            pl.BlockSpec((1, H), lambda i: (0, 0)),
            pl.BlockSpec((1, 1), lambda i: (0, 0)),
        ],
        out_specs=[
            pl.BlockSpec((R, H), lambda i: (i, 0)),
            pl.BlockSpec((1, 1, R), lambda i: (i, 0, 0)),
            pl.BlockSpec((1, 1, R), lambda i: (i, 0, 0)),
        ],
        out_shape=[
            jax.ShapeDtypeStruct((NP, H), jnp.float32),
            jax.ShapeDtypeStruct((G, 1, R), jnp.float32),
            jax.ShapeDtypeStruct((G, 1, R), jnp.float32),
        ],
    )(aggp, aggp, xpad, Wr1, Wo1, b1r, wpr1, wpo1, bp1s)


def _k3_body(a_hbm, srcp, dstp, out, sidx, didx, vals, zb, score_sh, sem):
    cid = lax.axis_index("c")
    sid = lax.axis_index("s")
    w = cid * NS + sid

    @pl.when(sid == 0)
    def _zero():
        def zbody(i, carry):
            zb[pl.ds(i * 16, 16)] = jnp.zeros((16,), jnp.float32)
            return carry
        lax.fori_loop(0, NP // 16, zbody, 0)
        pltpu.sync_copy(zb, score_sh)

    pltpu.sync_copy(srcp.at[pl.ds(w * PPW, PPW)], sidx)
    pltpu.sync_copy(dstp.at[pl.ds(w * PPW, PPW)], didx)

    for p in range(PPW):
        def gfire(j, carry):
            pltpu.async_copy(a_hbm.at[sidx.at[p].at[j]], vals.at[p].at[j],
                             sem.at[0])
            return carry
        lax.fori_loop(0, RPP, gfire, 0)
    for p in range(PPW):
        def gdrain(j, carry):
            pltpu.make_async_copy(a_hbm.at[sidx.at[p].at[j]],
                                  vals.at[p].at[j], sem.at[0]).wait()
            return carry
        lax.fori_loop(0, RPP, gdrain, 0)
    plsc.subcore_barrier()

    for p in range(PPW):
        def sfire(j, carry):
            pltpu.async_copy(vals.at[p].at[j], score_sh.at[didx.at[p].at[j]],
                             sem.at[1], add=True)
            return carry
        lax.fori_loop(0, RPP, sfire, 0)
    for p in range(PPW):
        def sdrain(j, carry):
            pltpu.make_async_copy(vals.at[p].at[j],
                                  score_sh.at[didx.at[p].at[j]],
                                  sem.at[1]).wait()
            return carry
        lax.fori_loop(0, RPP, sdrain, 0)
    plsc.subcore_barrier()

    @pl.when(sid == 0)
    def _out():
        pltpu.sync_copy(score_sh, zb)
        pltpu.sync_copy(zb, out.at[cid])


_k3 = functools.partial(
    pl.kernel,
    out_type=jax.ShapeDtypeStruct((NC, NP), jnp.float32),
    mesh=plsc.VectorSubcoreMesh(core_axis_name="c", subcore_axis_name="s",
                                num_cores=NC, num_subcores=NS),
    scratch_types=[
        pltpu.VMEM((PPW, RPP, 128), jnp.int32),
        pltpu.VMEM((PPW, RPP, 128), jnp.int32),
        pltpu.VMEM((PPW, RPP, 128), jnp.float32),
        pltpu.VMEM((NP,), jnp.float32),
        pltpu.VMEM_SHARED((NP,), jnp.float32),
        pltpu.SemaphoreType.DMA((2,)),
    ],
)(_k3_body)


def _k4_body(scorep, bvec, h, edges,
             wr2, wo2, b2r, wpr2, wpo2, bp2s,
             wr3, wo3, b3r, wpr3, wpo3, bp3s,
             wmt, bmr, out):
    s = scorep[0:1, :] + scorep[1:2, :] + bvec[...]
    iota = lax.broadcasted_iota(jnp.int32, (1, NP), 1)
    s = jnp.where(iota < N, s, jnp.float32(-3.0e38))
    m = jnp.max(s)
    idx = jnp.min(jnp.where(s == m, iota, NP))
    xn = h[pl.ds(idx, 1), :] * jnp.tanh(m)
    e0 = edges[0]
    e1 = edges[1]
    cf = jnp.sum(jnp.where((e0 == idx) & (e1 == idx), 1.0, 0.0))

    def gconv(v, wr, wo, br):
        y = cf * jnp.dot(v, wr[...], preferred_element_type=jnp.float32)
        y += jnp.dot(v, wo[...], preferred_element_type=jnp.float32)
        return jnp.maximum(y + br[...], 0.0)

    g2 = gconv(xn, wr2, wo2, b2r)
    s2 = cf * jnp.sum(g2 * wpr2[...]) + jnp.sum(g2 * wpo2[...]) + bp2s[0, 0]
    xn2 = g2 * jnp.tanh(s2)
    g3 = gconv(xn2, wr3, wo3, b3r)
    s3 = cf * jnp.sum(g3 * wpr3[...]) + jnp.sum(g3 * wpo3[...]) + bp3s[0, 0]
    xn3 = g3 * jnp.tanh(s3)
    t = xn + xn2 + xn3
    o0 = jnp.sum(t * wmt[0:1, :]) + bmr[0, 0]
    o1 = jnp.sum(t * wmt[1:2, :]) + bmr[0, 1]
    out[...] = jnp.concatenate([o0.reshape(1, 1), o1.reshape(1, 1)], axis=1)


def _k4(scorep, bvec, h, edges, *ws):
    return pl.pallas_call(
        _k4_body,
        out_shape=jax.ShapeDtypeStruct((1, 2), jnp.float32),
    )(scorep, bvec, h, edges, *ws)


def kernel(x, edge_index, batch, Wr1, Wo1, b1, Wpr1, Wpo1, bp1,
           Wr2, Wo2, b2, Wpr2, Wpo2, bp2, Wr3, Wo3, b3, Wpr3, Wpo3, bp3,
           Wm, bm):
    src = edge_index[0]
    dst = edge_index[1]
    pad = jnp.full((EP - E,), N, jnp.int32)
    srcp = jnp.concatenate([src.astype(jnp.int32), pad]).reshape(
        NW * PPW, RPP, 128)
    dstp = jnp.concatenate([dst.astype(jnp.int32), pad]).reshape(
        NW * PPW, RPP, 128)
    xpad = jnp.concatenate([x, jnp.zeros((NP - N, F), jnp.float32)], axis=0)
    xsplit = jnp.concatenate([xpad[:, :FH], xpad[:, FH:]], axis=0)
    zeros2d = jnp.zeros((NP, FH), jnp.float32)

    aggp = _k1(xsplit, srcp, dstp, zeros2d)

    h, a3, b3v = _k2(aggp, xpad,
                     Wr1, Wo1, b1.reshape(1, H),
                     Wpr1.reshape(1, H), Wpo1.reshape(1, H),
                     bp1.reshape(1, 1))
    a1 = a3.reshape(NP)
    bvec = b3v.reshape(1, NP)

    scorep = _k3(a1, srcp, dstp)

    edges = edge_index.astype(jnp.int32).reshape(2, E // 128, 128)
    wmt = (Wm[:H] + Wm[H:]).T
    return _k4(scorep, bvec, h, edges,
               Wr2, Wo2, b2.reshape(1, H), Wpr2.reshape(1, H),
               Wpo2.reshape(1, H), bp2.reshape(1, 1),
               Wr3, Wo3, b3.reshape(1, H), Wpr3.reshape(1, H),
               Wpo3.reshape(1, H), bp3.reshape(1, 1),
               wmt, bm.reshape(1, 2))


# speedup vs baseline: 2.6843x; 1.2521x over previous
import functools

import jax
import jax.numpy as jnp
from jax import lax
from jax.experimental import pallas as pl
from jax.experimental.pallas import tpu as pltpu
from jax.experimental.pallas import tpu_sc as plsc

N = 10000
E = 320000
F = 128
H = 256
NP = 10240
NC, NS = 2, 16
NW = NC * NS
RPP = 40
PPW = 2
EP = NW * PPW * RPP * 128
ROWS_PER_TILE = NP // NS


FH = F // 2
TPP = 4


def _k1_body(xsplit, srcp, dstp, zeros2d, out, sidx, didx, rows,
             x_sh, agg_sh, gsem, ssem):
    cid = lax.axis_index("c")
    sid = lax.axis_index("s")
    r_base = sid * ROWS_PER_TILE
    x_half = xsplit.at[pl.ds(cid * NP, NP)]
    pltpu.sync_copy(x_half.at[pl.ds(r_base, ROWS_PER_TILE)],
                    x_sh.at[pl.ds(r_base, ROWS_PER_TILE)])
    pltpu.sync_copy(zeros2d.at[pl.ds(r_base, ROWS_PER_TILE)],
                    agg_sh.at[pl.ds(r_base, ROWS_PER_TILE)])
    plsc.subcore_barrier()

    NB = 4
    for p in range(TPP):
        pltpu.sync_copy(srcp.at[sid * TPP + p], sidx)
        pltpu.sync_copy(dstp.at[sid * TPP + p], didx)
        for b in range(2):
            pltpu.async_copy(x_sh.at[sidx.at[b]], rows.at[b], gsem.at[b])

        def body(t, carry):
            for u in range(NB):
                j = t * NB + u
                b = j % NB
                pltpu.make_async_copy(x_sh.at[sidx.at[j]], rows.at[b],
                                      gsem.at[b]).wait()
                pltpu.async_copy(rows.at[b], agg_sh.at[didx.at[j]],
                                 ssem.at[b], add=True)
                b2 = (j + 2) % NB

                @pl.when(j >= 2)
                def _():
                    pltpu.make_async_copy(rows.at[b2],
                                          agg_sh.at[didx.at[j - 2]],
                                          ssem.at[b2]).wait()

                @pl.when(j + 2 < RPP)
                def _():
                    pltpu.async_copy(x_sh.at[sidx.at[j + 2]], rows.at[b2],
                                     gsem.at[b2])
            return carry

        lax.fori_loop(0, RPP // NB, body, 0)
        for j in (RPP - 2, RPP - 1):
            pltpu.make_async_copy(rows.at[j % NB], agg_sh.at[didx.at[j]],
                                  ssem.at[j % NB]).wait()
    plsc.subcore_barrier()

    def body2(t, carry):
        r0 = r_base + t * 128
        pltpu.sync_copy(agg_sh.at[pl.ds(r0, 128)], rows.at[0])
        pltpu.sync_copy(rows.at[0], out.at[pl.ds(cid * NP + r0, 128)])
        return carry

    lax.fori_loop(0, ROWS_PER_TILE // 128, body2, 0)


_k1 = functools.partial(
    pl.kernel,
    out_type=jax.ShapeDtypeStruct((NC * NP, FH), jnp.float32),
    mesh=plsc.VectorSubcoreMesh(core_axis_name="c", subcore_axis_name="s",
                                num_cores=NC, num_subcores=NS),
    scratch_types=[
        pltpu.VMEM((RPP, 128), jnp.int32),
        pltpu.VMEM((RPP, 128), jnp.int32),
        pltpu.VMEM((4, 128, FH), jnp.float32),
        pltpu.VMEM_SHARED((NP, FH), jnp.float32),
        pltpu.VMEM_SHARED((NP, FH), jnp.float32),
        pltpu.SemaphoreType.DMA((4,)),
        pltpu.SemaphoreType.DMA((4,)),
    ],
    compiler_params=pltpu.CompilerParams(use_tc_tiling_on_sc=False),
)(_k1_body)


def _k2_body(agg0, agg1, xb, wr, wo, b1r, wpr, wpo, bp1s, h_out, a_out, b_out):
    aggb = jnp.concatenate([agg0[...], agg1[...]], axis=1)
    h = jnp.dot(aggb, wr[...], preferred_element_type=jnp.float32)
    h += jnp.dot(xb[...], wo[...], preferred_element_type=jnp.float32)
    h = jnp.maximum(h + b1r[...], 0.0)
    h_out[...] = h
    a_out[...] = jnp.sum(h * wpr[...], axis=1).reshape(1, 1, -1)
    b_out[...] = (jnp.sum(h * wpo[...], axis=1) + bp1s[0, 0]).reshape(1, 1, -1)


def _k2(aggp, xpad, Wr1, Wo1, b1r, wpr1, wpo1, bp1s):
    R = 1024
    G = NP // R
    return pl.pallas_call(
        _k2_body,
        grid=(G,),
        in_specs=[
            pl.BlockSpec((R, FH), lambda i: (i, 0)),
            pl.BlockSpec((R, FH), lambda i: (i + G, 0)),
            pl.BlockSpec((R, F), lambda i: (i, 0)),
            pl.BlockSpec((F, H), lambda i: (0, 0)),
            pl.BlockSpec((F, H), lambda i: (0, 0)),
            pl.BlockSpec((1, H), lambda i: (0, 0)),
            pl.BlockSpec((1, H), lambda i: (0, 0)),
            pl.BlockSpec((1, H), lambda i: (0, 0)),
            pl.BlockSpec((1, 1), lambda i: (0, 0)),
        ],
        out_specs=[
            pl.BlockSpec((R, H), lambda i: (i, 0)),
            pl.BlockSpec((1, 1, R), lambda i: (i, 0, 0)),
            pl.BlockSpec((1, 1, R), lambda i: (i, 0, 0)),
        ],
        out_shape=[
            jax.ShapeDtypeStruct((NP, H), jnp.float32),
            jax.ShapeDtypeStruct((G, 1, R), jnp.float32),
            jax.ShapeDtypeStruct((G, 1, R), jnp.float32),
        ],
    )(aggp, aggp, xpad, Wr1, Wo1, b1r, wpr1, wpo1, bp1s)


def _k3_body(a_hbm, srcp, dstp, out, sidx, didx, vals, zb, score_sh, a_sh,
             sem):
    cid = lax.axis_index("c")
    sid = lax.axis_index("s")
    w = cid * NS + sid

    @pl.when(sid == 0)
    def _zero():
        def zbody(i, carry):
            zb[pl.ds(i * 16, 16)] = jnp.zeros((16,), jnp.float32)
            return carry
        lax.fori_loop(0, NP // 16, zbody, 0)
        pltpu.sync_copy(zb, score_sh)
        pltpu.sync_copy(a_hbm, a_sh)

    pltpu.sync_copy(srcp.at[pl.ds(w * PPW, PPW)], sidx)
    pltpu.sync_copy(dstp.at[pl.ds(w * PPW, PPW)], didx)
    plsc.subcore_barrier()

    for p in range(PPW):
        def gfire(j, carry):
            pltpu.async_copy(a_sh.at[sidx.at[p].at[j]], vals.at[p].at[j],
                             sem.at[0])
            return carry
        lax.fori_loop(0, RPP, gfire, 0)
    for p in range(PPW):
        def gdrain(j, carry):
            pltpu.make_async_copy(a_sh.at[sidx.at[p].at[j]],
                                  vals.at[p].at[j], sem.at[0]).wait()
            return carry
        lax.fori_loop(0, RPP, gdrain, 0)

    for p in range(PPW):
        def sfire(j, carry):
            pltpu.async_copy(vals.at[p].at[j], score_sh.at[didx.at[p].at[j]],
                             sem.at[1], add=True)
            return carry
        lax.fori_loop(0, RPP, sfire, 0)
    for p in range(PPW):
        def sdrain(j, carry):
            pltpu.make_async_copy(vals.at[p].at[j],
                                  score_sh.at[didx.at[p].at[j]],
                                  sem.at[1]).wait()
            return carry
        lax.fori_loop(0, RPP, sdrain, 0)
    plsc.subcore_barrier()

    @pl.when(sid == 0)
    def _out():
        pltpu.sync_copy(score_sh, zb)
        pltpu.sync_copy(zb, out.at[cid])


_k3 = functools.partial(
    pl.kernel,
    out_type=jax.ShapeDtypeStruct((NC, NP), jnp.float32),
    mesh=plsc.VectorSubcoreMesh(core_axis_name="c", subcore_axis_name="s",
                                num_cores=NC, num_subcores=NS),
    scratch_types=[
        pltpu.VMEM((PPW, RPP, 128), jnp.int32),
        pltpu.VMEM((PPW, RPP, 128), jnp.int32),
        pltpu.VMEM((PPW, RPP, 128), jnp.float32),
        pltpu.VMEM((NP,), jnp.float32),
        pltpu.VMEM_SHARED((NP,), jnp.float32),
        pltpu.VMEM_SHARED((NP,), jnp.float32),
        pltpu.SemaphoreType.DMA((2,)),
    ],
    compiler_params=pltpu.CompilerParams(use_tc_tiling_on_sc=False),
)(_k3_body)


def _k4_body(scorep, bvec, h, edges,
             wr2, wo2, b2r, wpr2, wpo2, bp2s,
             wr3, wo3, b3r, wpr3, wpo3, bp3s,
             wmt, bmr, out):
    s = scorep[0:1, :] + scorep[1:2, :] + bvec[...]
    iota = lax.broadcasted_iota(jnp.int32, (1, NP), 1)
    s = jnp.where(iota < N, s, jnp.float32(-3.0e38))
    m = jnp.max(s)
    idx = jnp.min(jnp.where(s == m, iota, NP))
    xn = h[pl.ds(idx, 1), :] * jnp.tanh(m)
    e0 = edges[0]
    e1 = edges[1]
    cf = jnp.sum(jnp.where((e0 == idx) & (e1 == idx), 1.0, 0.0))

    def gconv(v, wr, wo, br):
        y = cf * jnp.dot(v, wr[...], preferred_element_type=jnp.float32)
        y += jnp.dot(v, wo[...], preferred_element_type=jnp.float32)
        return jnp.maximum(y + br[...], 0.0)

    g2 = gconv(xn, wr2, wo2, b2r)
    s2 = cf * jnp.sum(g2 * wpr2[...]) + jnp.sum(g2 * wpo2[...]) + bp2s[0, 0]
    xn2 = g2 * jnp.tanh(s2)
    g3 = gconv(xn2, wr3, wo3, b3r)
    s3 = cf * jnp.sum(g3 * wpr3[...]) + jnp.sum(g3 * wpo3[...]) + bp3s[0, 0]
    xn3 = g3 * jnp.tanh(s3)
    t = xn + xn2 + xn3
    o0 = jnp.sum(t * wmt[0:1, :]) + bmr[0, 0]
    o1 = jnp.sum(t * wmt[1:2, :]) + bmr[0, 1]
    out[...] = jnp.concatenate([o0.reshape(1, 1), o1.reshape(1, 1)], axis=1)


def _k4(scorep, bvec, h, edges, *ws):
    return pl.pallas_call(
        _k4_body,
        out_shape=jax.ShapeDtypeStruct((1, 2), jnp.float32),
    )(scorep, bvec, h, edges, *ws)


def kernel(x, edge_index, batch, Wr1, Wo1, b1, Wpr1, Wpo1, bp1,
           Wr2, Wo2, b2, Wpr2, Wpo2, bp2, Wr3, Wo3, b3, Wpr3, Wpo3, bp3,
           Wm, bm):
    src = edge_index[0]
    dst = edge_index[1]
    pad = jnp.full((EP - E,), N, jnp.int32)
    srcp = jnp.concatenate([src.astype(jnp.int32), pad]).reshape(
        NW * PPW, RPP, 128)
    dstp = jnp.concatenate([dst.astype(jnp.int32), pad]).reshape(
        NW * PPW, RPP, 128)
    xpad = jnp.concatenate([x, jnp.zeros((NP - N, F), jnp.float32)], axis=0)
    xsplit = jnp.concatenate([xpad[:, :FH], xpad[:, FH:]], axis=0)
    zeros2d = jnp.zeros((NP, FH), jnp.float32)

    aggp = _k1(xsplit, srcp, dstp, zeros2d)

    h, a3, b3v = _k2(aggp, xpad,
                     Wr1, Wo1, b1.reshape(1, H),
                     Wpr1.reshape(1, H), Wpo1.reshape(1, H),
                     bp1.reshape(1, 1))
    a1 = a3.reshape(NP)
    bvec = b3v.reshape(1, NP)

    scorep = _k3(a1, srcp, dstp)

    edges = edge_index.astype(jnp.int32).reshape(2, E // 128, 128)
    wmt = (Wm[:H] + Wm[H:]).T
    return _k4(scorep, bvec, h, edges,
               Wr2, Wo2, b2.reshape(1, H), Wpr2.reshape(1, H),
               Wpo2.reshape(1, H), bp2.reshape(1, 1),
               Wr3, Wo3, b3.reshape(1, H), Wpr3.reshape(1, H),
               Wpo3.reshape(1, H), bp3.reshape(1, 1),
               wmt, bm.reshape(1, 2))


# speedup vs baseline: 2.9314x; 1.0920x over previous
import functools

import jax
import jax.numpy as jnp
from jax import lax
from jax.experimental import pallas as pl
from jax.experimental.pallas import tpu as pltpu
from jax.experimental.pallas import tpu_sc as plsc

N = 10000
E = 320000
F = 128
H = 256
NP = 10240
NC, NS = 2, 16
NW = NC * NS
RPP = 40
PPW = 2
EP = NW * PPW * RPP * 128
ROWS_PER_TILE = NP // NS


FH = F // 2
TPP = 4


def _k1_body(x, srcp, dstp, zeros2d, out, sidx, didx, rows,
             x_sh, agg_sh, gsem, ssem):
    cid = lax.axis_index("c")
    sid = lax.axis_index("s")
    r_base = sid * ROWS_PER_TILE
    @pl.when(sid < NS - 1)
    def _():
        pltpu.sync_copy(x.at[pl.ds(r_base, ROWS_PER_TILE),
                             pl.ds(cid * FH, FH)],
                        x_sh.at[pl.ds(r_base, ROWS_PER_TILE)])
        pltpu.sync_copy(zeros2d.at[pl.ds(r_base, ROWS_PER_TILE)],
                        agg_sh.at[pl.ds(r_base, ROWS_PER_TILE)])

    @pl.when(sid == NS - 1)
    def _():
        last = N - (NS - 1) * ROWS_PER_TILE
        pltpu.sync_copy(x.at[pl.ds(r_base, last), pl.ds(cid * FH, FH)],
                        x_sh.at[pl.ds(r_base, last)])
        pltpu.sync_copy(zeros2d.at[pl.ds(r_base, last)],
                        agg_sh.at[pl.ds(r_base, last)])

    plsc.subcore_barrier()

    NB = 4
    for p in range(TPP):
        pltpu.sync_copy(srcp.at[sid * TPP + p], sidx)
        pltpu.sync_copy(dstp.at[sid * TPP + p], didx)
        for b in range(2):
            pltpu.async_copy(x_sh.at[sidx.at[b]], rows.at[b], gsem.at[b])

        def body(t, carry):
            for u in range(NB):
                j = t * NB + u
                b = j % NB
                pltpu.make_async_copy(x_sh.at[sidx.at[j]], rows.at[b],
                                      gsem.at[b]).wait()
                pltpu.async_copy(rows.at[b], agg_sh.at[didx.at[j]],
                                 ssem.at[b], add=True)
                b2 = (j + 2) % NB

                @pl.when(j >= 2)
                def _():
                    pltpu.make_async_copy(rows.at[b2],
                                          agg_sh.at[didx.at[j - 2]],
                                          ssem.at[b2]).wait()

                @pl.when(j + 2 < RPP)
                def _():
                    pltpu.async_copy(x_sh.at[sidx.at[j + 2]], rows.at[b2],
                                     gsem.at[b2])
            return carry

        lax.fori_loop(0, RPP // NB, body, 0)
        for j in (RPP - 2, RPP - 1):
            pltpu.make_async_copy(rows.at[j % NB], agg_sh.at[didx.at[j]],
                                  ssem.at[j % NB]).wait()
    plsc.subcore_barrier()

    def body2(t, carry):
        r0 = r_base + t * 128
        pltpu.sync_copy(agg_sh.at[pl.ds(r0, 128)], rows.at[0])
        pltpu.sync_copy(rows.at[0], out.at[cid].at[pl.ds(r0, 128)])
        return carry

    lax.fori_loop(0, ROWS_PER_TILE // 128, body2, 0)


_k1 = functools.partial(
    pl.kernel,
    out_type=jax.ShapeDtypeStruct((NC, NP, FH), jnp.float32),
    mesh=plsc.VectorSubcoreMesh(core_axis_name="c", subcore_axis_name="s",
                                num_cores=NC, num_subcores=NS),
    scratch_types=[
        pltpu.VMEM((RPP, 128), jnp.int32),
        pltpu.VMEM((RPP, 128), jnp.int32),
        pltpu.VMEM((4, 128, FH), jnp.float32),
        pltpu.VMEM_SHARED((NP, FH), jnp.float32),
        pltpu.VMEM_SHARED((NP, FH), jnp.float32),
        pltpu.SemaphoreType.DMA((4,)),
        pltpu.SemaphoreType.DMA((4,)),
    ],
    compiler_params=pltpu.CompilerParams(use_tc_tiling_on_sc=False),
)(_k1_body)


def _k2_body(agg0, agg1, xb, wr, wo, b1r, wpr, wpo, bp1s, h_out, a_out, b_out):
    aggb = jnp.concatenate([agg0[0], agg1[0]], axis=1)
    h = jnp.dot(aggb, wr[...], preferred_element_type=jnp.float32)
    h += jnp.dot(xb[...], wo[...], preferred_element_type=jnp.float32)
    h = jnp.maximum(h + b1r[...], 0.0)
    h_out[...] = h
    a_out[...] = jnp.sum(h * wpr[...], axis=1).reshape(1, 1, -1)
    b_out[...] = (jnp.sum(h * wpo[...], axis=1) + bp1s[0, 0]).reshape(1, 1, -1)


def _k2(aggp, x, Wr1, Wo1, b1r, wpr1, wpo1, bp1s):
    R = 1000
    G = N // R
    return pl.pallas_call(
        _k2_body,
        grid=(G,),
        in_specs=[
            pl.BlockSpec((1, R, FH), lambda i: (0, i, 0)),
            pl.BlockSpec((1, R, FH), lambda i: (1, i, 0)),
            pl.BlockSpec((R, F), lambda i: (i, 0)),
            pl.BlockSpec((F, H), lambda i: (0, 0)),
            pl.BlockSpec((F, H), lambda i: (0, 0)),
            pl.BlockSpec((1, H), lambda i: (0, 0)),
            pl.BlockSpec((1, H), lambda i: (0, 0)),
            pl.BlockSpec((1, H), lambda i: (0, 0)),
            pl.BlockSpec((1, 1), lambda i: (0, 0)),
        ],
        out_specs=[
            pl.BlockSpec((R, H), lambda i: (i, 0)),
            pl.BlockSpec((1, 1, R), lambda i: (i, 0, 0)),
            pl.BlockSpec((1, 1, R), lambda i: (i, 0, 0)),
        ],
        out_shape=[
            jax.ShapeDtypeStruct((N, H), jnp.float32),
            jax.ShapeDtypeStruct((G, 1, R), jnp.float32),
            jax.ShapeDtypeStruct((G, 1, R), jnp.float32),
        ],
    )(aggp, aggp, x, Wr1, Wo1, b1r, wpr1, wpo1, bp1s)


def _k3_body(a_hbm, srcp, dstp, out, sidx, didx, vals, zb, score_sh, a_sh,
             sem):
    cid = lax.axis_index("c")
    sid = lax.axis_index("s")
    w = cid * NS + sid

    @pl.when(sid == 0)
    def _zero():
        def zbody(i, carry):
            zb[pl.ds(i * 16, 16)] = jnp.zeros((16,), jnp.float32)
            return carry
        lax.fori_loop(0, NP // 16, zbody, 0)
        pltpu.sync_copy(zb, score_sh)
        pltpu.sync_copy(a_hbm, a_sh.at[pl.ds(0, N)])

    pltpu.sync_copy(srcp.at[pl.ds(w * PPW, PPW)], sidx)
    pltpu.sync_copy(dstp.at[pl.ds(w * PPW, PPW)], didx)
    plsc.subcore_barrier()

    for p in range(PPW):
        def gfire(j, carry):
            pltpu.async_copy(a_sh.at[sidx.at[p].at[j]], vals.at[p].at[j],
                             sem.at[0])
            return carry
        lax.fori_loop(0, RPP, gfire, 0)
    for p in range(PPW):
        def gdrain(j, carry):
            pltpu.make_async_copy(a_sh.at[sidx.at[p].at[j]],
                                  vals.at[p].at[j], sem.at[0]).wait()
            return carry
        lax.fori_loop(0, RPP, gdrain, 0)

    for p in range(PPW):
        def sfire(j, carry):
            pltpu.async_copy(vals.at[p].at[j], score_sh.at[didx.at[p].at[j]],
                             sem.at[1], add=True)
            return carry
        lax.fori_loop(0, RPP, sfire, 0)
    for p in range(PPW):
        def sdrain(j, carry):
            pltpu.make_async_copy(vals.at[p].at[j],
                                  score_sh.at[didx.at[p].at[j]],
                                  sem.at[1]).wait()
            return carry
        lax.fori_loop(0, RPP, sdrain, 0)
    plsc.subcore_barrier()

    @pl.when(sid == 0)
    def _out():
        pltpu.sync_copy(score_sh, zb)
        pltpu.sync_copy(zb, out.at[cid])


_k3 = functools.partial(
    pl.kernel,
    out_type=jax.ShapeDtypeStruct((NC, NP), jnp.float32),
    mesh=plsc.VectorSubcoreMesh(core_axis_name="c", subcore_axis_name="s",
                                num_cores=NC, num_subcores=NS),
    scratch_types=[
        pltpu.VMEM((PPW, RPP, 128), jnp.int32),
        pltpu.VMEM((PPW, RPP, 128), jnp.int32),
        pltpu.VMEM((PPW, RPP, 128), jnp.float32),
        pltpu.VMEM((NP,), jnp.float32),
        pltpu.VMEM_SHARED((NP,), jnp.float32),
        pltpu.VMEM_SHARED((NP,), jnp.float32),
        pltpu.SemaphoreType.DMA((2,)),
    ],
    compiler_params=pltpu.CompilerParams(use_tc_tiling_on_sc=False),
)(_k3_body)


def _k4_body(scorep, bvec, h, edges,
             wr2, wo2, b2r, wpr2, wpo2, bp2s,
             wr3, wo3, b3r, wpr3, wpo3, bp3s,
             wmt, bmr, out):
    s = scorep[0:1, :N] + scorep[1:2, :N] + bvec[...]
    iota = lax.broadcasted_iota(jnp.int32, (1, N), 1)
    m = jnp.max(s)
    idx = jnp.min(jnp.where(s == m, iota, N))
    xn = h[pl.ds(idx, 1), :] * jnp.tanh(m)
    e0 = edges[0]
    e1 = edges[1]
    cf = jnp.sum(jnp.where((e0 == idx) & (e1 == idx), 1.0, 0.0))

    def gconv(v, wr, wo, br):
        y = cf * jnp.dot(v, wr[...], preferred_element_type=jnp.float32)
        y += jnp.dot(v, wo[...], preferred_element_type=jnp.float32)
        return jnp.maximum(y + br[...], 0.0)

    g2 = gconv(xn, wr2, wo2, b2r)
    s2 = cf * jnp.sum(g2 * wpr2[...]) + jnp.sum(g2 * wpo2[...]) + bp2s[0, 0]
    xn2 = g2 * jnp.tanh(s2)
    g3 = gconv(xn2, wr3, wo3, b3r)
    s3 = cf * jnp.sum(g3 * wpr3[...]) + jnp.sum(g3 * wpo3[...]) + bp3s[0, 0]
    xn3 = g3 * jnp.tanh(s3)
    t = xn + xn2 + xn3
    o0 = jnp.sum(t * wmt[0:1, :]) + bmr[0, 0]
    o1 = jnp.sum(t * wmt[1:2, :]) + bmr[0, 1]
    out[...] = jnp.concatenate([o0.reshape(1, 1), o1.reshape(1, 1)], axis=1)


def _k4(scorep, bvec, h, edges, *ws):
    return pl.pallas_call(
        _k4_body,
        out_shape=jax.ShapeDtypeStruct((1, 2), jnp.float32),
    )(scorep, bvec, h, edges, *ws)


def kernel(x, edge_index, batch, Wr1, Wo1, b1, Wpr1, Wpo1, bp1,
           Wr2, Wo2, b2, Wpr2, Wpo2, bp2, Wr3, Wo3, b3, Wpr3, Wpo3, bp3,
           Wm, bm):
    src = edge_index[0]
    dst = edge_index[1]
    spad = jnp.zeros((EP - E,), jnp.int32)
    dpad = jnp.full((EP - E,), N, jnp.int32)
    srcp = jnp.concatenate([src.astype(jnp.int32), spad]).reshape(
        NW * PPW, RPP, 128)
    dstp = jnp.concatenate([dst.astype(jnp.int32), dpad]).reshape(
        NW * PPW, RPP, 128)
    zeros2d = jnp.zeros((N, FH), jnp.float32)

    aggp = _k1(x, srcp, dstp, zeros2d)

    h, a3, b3v = _k2(aggp, x,
                     Wr1, Wo1, b1.reshape(1, H),
                     Wpr1.reshape(1, H), Wpo1.reshape(1, H),
                     bp1.reshape(1, 1))
    a1 = a3.reshape(N)
    bvec = b3v.reshape(1, N)

    scorep = _k3(a1, srcp, dstp)

    edges = edge_index.astype(jnp.int32).reshape(2, E // 128, 128)
    wmt = (Wm[:H] + Wm[H:]).T
    return _k4(scorep, bvec, h, edges,
               Wr2, Wo2, b2.reshape(1, H), Wpr2.reshape(1, H),
               Wpo2.reshape(1, H), bp2.reshape(1, 1),
               Wr3, Wo3, b3.reshape(1, H), Wpr3.reshape(1, H),
               Wpo3.reshape(1, H), bp3.reshape(1, 1),
               wmt, bm.reshape(1, 2))


# speedup vs baseline: 2.9570x; 1.0088x over previous
import functools

import jax
import jax.numpy as jnp
from jax import lax
from jax.experimental import pallas as pl
from jax.experimental.pallas import tpu as pltpu
from jax.experimental.pallas import tpu_sc as plsc

N = 10000
E = 320000
F = 128
H = 256
NP = 10240
NC, NS = 2, 16
NW = NC * NS
RPP = 40
PPW = 2
EP = NW * PPW * RPP * 128
ROWS_PER_TILE = NP // NS


FH = F // 2
TPP = 4


def _k1_body(x, srcp, dstp, zeros2d, out, sidx, didx, rows,
             x_sh, agg_sh, gsem, ssem):
    cid = lax.axis_index("c")
    sid = lax.axis_index("s")
    r_base = sid * ROWS_PER_TILE
    @pl.when(sid < NS - 1)
    def _():
        pltpu.sync_copy(x.at[pl.ds(r_base, ROWS_PER_TILE),
                             pl.ds(cid * FH, FH)],
                        x_sh.at[pl.ds(r_base, ROWS_PER_TILE)])
        pltpu.sync_copy(zeros2d.at[pl.ds(r_base, ROWS_PER_TILE)],
                        agg_sh.at[pl.ds(r_base, ROWS_PER_TILE)])

    @pl.when(sid == NS - 1)
    def _():
        last = N - (NS - 1) * ROWS_PER_TILE
        pltpu.sync_copy(x.at[pl.ds(r_base, last), pl.ds(cid * FH, FH)],
                        x_sh.at[pl.ds(r_base, last)])
        pltpu.sync_copy(zeros2d.at[pl.ds(r_base, last)],
                        agg_sh.at[pl.ds(r_base, last)])

    plsc.subcore_barrier()

    NB = 4
    for p in range(TPP):
        pltpu.sync_copy(srcp.at[sid * TPP + p], sidx)
        pltpu.sync_copy(dstp.at[sid * TPP + p], didx)
        for b in range(2):
            pltpu.async_copy(x_sh.at[sidx.at[b]], rows.at[b], gsem.at[b])

        def body(t, carry):
            for u in range(NB):
                j = t * NB + u
                b = j % NB
                pltpu.make_async_copy(x_sh.at[sidx.at[j]], rows.at[b],
                                      gsem.at[b]).wait()
                pltpu.async_copy(rows.at[b], agg_sh.at[didx.at[j]],
                                 ssem.at[b], add=True)
                b2 = (j + 2) % NB

                @pl.when(j >= 2)
                def _():
                    pltpu.make_async_copy(rows.at[b2],
                                          agg_sh.at[didx.at[j - 2]],
                                          ssem.at[b2]).wait()

                @pl.when(j + 2 < RPP)
                def _():
                    pltpu.async_copy(x_sh.at[sidx.at[j + 2]], rows.at[b2],
                                     gsem.at[b2])
            return carry

        lax.fori_loop(0, RPP // NB, body, 0)
        for j in (RPP - 2, RPP - 1):
            pltpu.make_async_copy(rows.at[j % NB], agg_sh.at[didx.at[j]],
                                  ssem.at[j % NB]).wait()
    plsc.subcore_barrier()

    def body2(t, carry):
        r0 = r_base + t * 128
        pltpu.sync_copy(agg_sh.at[pl.ds(r0, 128)], rows.at[0])
        pltpu.sync_copy(rows.at[0], out.at[cid].at[pl.ds(r0, 128)])
        return carry

    lax.fori_loop(0, ROWS_PER_TILE // 128, body2, 0)


_k1 = functools.partial(
    pl.kernel,
    out_type=jax.ShapeDtypeStruct((NC, NP, FH), jnp.float32),
    mesh=plsc.VectorSubcoreMesh(core_axis_name="c", subcore_axis_name="s",
                                num_cores=NC, num_subcores=NS),
    scratch_types=[
        pltpu.VMEM((RPP, 128), jnp.int32),
        pltpu.VMEM((RPP, 128), jnp.int32),
        pltpu.VMEM((4, 128, FH), jnp.float32),
        pltpu.VMEM_SHARED((NP, FH), jnp.float32),
        pltpu.VMEM_SHARED((NP, FH), jnp.float32),
        pltpu.SemaphoreType.DMA((4,)),
        pltpu.SemaphoreType.DMA((4,)),
    ],
    compiler_params=pltpu.CompilerParams(use_tc_tiling_on_sc=False),
)(_k1_body)


def _k2_body(agg0, agg1, xb, wr, wo, b1r, wpr, wpo, bp1s, a_out, b_out):
    aggb = jnp.concatenate([agg0[0], agg1[0]], axis=1)
    h = jnp.dot(aggb, wr[...], preferred_element_type=jnp.float32)
    h += jnp.dot(xb[...], wo[...], preferred_element_type=jnp.float32)
    h = jnp.maximum(h + b1r[...], 0.0)
    a_out[...] = jnp.sum(h * wpr[...], axis=1).reshape(1, 1, -1)
    b_out[...] = (jnp.sum(h * wpo[...], axis=1) + bp1s[0, 0]).reshape(1, 1, -1)


def _k2(aggp, x, Wr1, Wo1, b1r, wpr1, wpo1, bp1s):
    R = 1000
    G = N // R
    return pl.pallas_call(
        _k2_body,
        grid=(G,),
        in_specs=[
            pl.BlockSpec((1, R, FH), lambda i: (0, i, 0)),
            pl.BlockSpec((1, R, FH), lambda i: (1, i, 0)),
            pl.BlockSpec((R, F), lambda i: (i, 0)),
            pl.BlockSpec((F, H), lambda i: (0, 0)),
            pl.BlockSpec((F, H), lambda i: (0, 0)),
            pl.BlockSpec((1, H), lambda i: (0, 0)),
            pl.BlockSpec((1, H), lambda i: (0, 0)),
            pl.BlockSpec((1, H), lambda i: (0, 0)),
            pl.BlockSpec((1, 1), lambda i: (0, 0)),
        ],
        out_specs=[
            pl.BlockSpec((1, 1, R), lambda i: (i, 0, 0)),
            pl.BlockSpec((1, 1, R), lambda i: (i, 0, 0)),
        ],
        out_shape=[
            jax.ShapeDtypeStruct((G, 1, R), jnp.float32),
            jax.ShapeDtypeStruct((G, 1, R), jnp.float32),
        ],
    )(aggp, aggp, x, Wr1, Wo1, b1r, wpr1, wpo1, bp1s)


def _k3_body(a_hbm, srcp, dstp, out, sidx, didx, vals, zb, score_sh, a_sh,
             sem):
    cid = lax.axis_index("c")
    sid = lax.axis_index("s")
    w = cid * NS + sid

    @pl.when(sid == 0)
    def _zero():
        def zbody(i, carry):
            zb[pl.ds(i * 16, 16)] = jnp.zeros((16,), jnp.float32)
            return carry
        lax.fori_loop(0, NP // 16, zbody, 0)
        pltpu.sync_copy(zb, score_sh)
        pltpu.sync_copy(a_hbm, a_sh.at[pl.ds(0, N)])

    pltpu.sync_copy(srcp.at[pl.ds(w * PPW, PPW)], sidx)
    pltpu.sync_copy(dstp.at[pl.ds(w * PPW, PPW)], didx)
    plsc.subcore_barrier()

    for p in range(PPW):
        def gfire(j, carry):
            pltpu.async_copy(a_sh.at[sidx.at[p].at[j]], vals.at[p].at[j],
                             sem.at[0])
            return carry
        lax.fori_loop(0, RPP, gfire, 0)
    for p in range(PPW):
        def gdrain(j, carry):
            pltpu.make_async_copy(a_sh.at[sidx.at[p].at[j]],
                                  vals.at[p].at[j], sem.at[0]).wait()
            return carry
        lax.fori_loop(0, RPP, gdrain, 0)

    for p in range(PPW):
        def sfire(j, carry):
            pltpu.async_copy(vals.at[p].at[j], score_sh.at[didx.at[p].at[j]],
                             sem.at[1], add=True)
            return carry
        lax.fori_loop(0, RPP, sfire, 0)
    for p in range(PPW):
        def sdrain(j, carry):
            pltpu.make_async_copy(vals.at[p].at[j],
                                  score_sh.at[didx.at[p].at[j]],
                                  sem.at[1]).wait()
            return carry
        lax.fori_loop(0, RPP, sdrain, 0)
    plsc.subcore_barrier()

    @pl.when(sid == 0)
    def _out():
        pltpu.sync_copy(score_sh, zb)
        pltpu.sync_copy(zb, out.at[cid])


_k3 = functools.partial(
    pl.kernel,
    out_type=jax.ShapeDtypeStruct((NC, NP), jnp.float32),
    mesh=plsc.VectorSubcoreMesh(core_axis_name="c", subcore_axis_name="s",
                                num_cores=NC, num_subcores=NS),
    scratch_types=[
        pltpu.VMEM((PPW, RPP, 128), jnp.int32),
        pltpu.VMEM((PPW, RPP, 128), jnp.int32),
        pltpu.VMEM((PPW, RPP, 128), jnp.float32),
        pltpu.VMEM((NP,), jnp.float32),
        pltpu.VMEM_SHARED((NP,), jnp.float32),
        pltpu.VMEM_SHARED((NP,), jnp.float32),
        pltpu.SemaphoreType.DMA((2,)),
    ],
    compiler_params=pltpu.CompilerParams(use_tc_tiling_on_sc=False),
)(_k3_body)


def _k4_body(scorep, bvec, aggp, x, edges,
             wr1, wo1, b1r,
             wr2, wo2, b2r, wpr2, wpo2, bp2s,
             wr3, wo3, b3r, wpr3, wpo3, bp3s,
             wmt, bmr, out, arow0, arow1, xrow, sem):
    s = scorep[0:1, :N] + scorep[1:2, :N] + bvec[...]
    iota = lax.broadcasted_iota(jnp.int32, (1, N), 1)
    m = jnp.max(s)
    idx = jnp.min(jnp.where(s == m, iota, N))
    c0 = pltpu.make_async_copy(aggp.at[0].at[pl.ds(idx, 1)], arow0, sem)
    c0.start()
    c1 = pltpu.make_async_copy(aggp.at[1].at[pl.ds(idx, 1)], arow1, sem)
    c1.start()
    c2 = pltpu.make_async_copy(x.at[pl.ds(idx, 1)], xrow, sem)
    c2.start()
    c0.wait()
    c1.wait()
    c2.wait()
    agg_row = jnp.concatenate([arow0[...], arow1[...]], axis=1)
    hrow = jnp.dot(agg_row, wr1[...], preferred_element_type=jnp.float32)
    hrow += jnp.dot(xrow[...], wo1[...], preferred_element_type=jnp.float32)
    hrow = jnp.maximum(hrow + b1r[...], 0.0)
    xn = hrow * jnp.tanh(m)
    e0 = edges[0]
    e1 = edges[1]
    cf = jnp.sum(jnp.where((e0 == idx) & (e1 == idx), 1.0, 0.0))

    def gconv(v, wr, wo, br):
        y = cf * jnp.dot(v, wr[...], preferred_element_type=jnp.float32)
        y += jnp.dot(v, wo[...], preferred_element_type=jnp.float32)
        return jnp.maximum(y + br[...], 0.0)

    g2 = gconv(xn, wr2, wo2, b2r)
    s2 = cf * jnp.sum(g2 * wpr2[...]) + jnp.sum(g2 * wpo2[...]) + bp2s[0, 0]
    xn2 = g2 * jnp.tanh(s2)
    g3 = gconv(xn2, wr3, wo3, b3r)
    s3 = cf * jnp.sum(g3 * wpr3[...]) + jnp.sum(g3 * wpo3[...]) + bp3s[0, 0]
    xn3 = g3 * jnp.tanh(s3)
    t = xn + xn2 + xn3
    o0 = jnp.sum(t * wmt[0:1, :]) + bmr[0, 0]
    o1 = jnp.sum(t * wmt[1:2, :]) + bmr[0, 1]
    out[...] = jnp.concatenate([o0.reshape(1, 1), o1.reshape(1, 1)], axis=1)


def _k4(scorep, bvec, aggp, x, edges, *ws):
    return pl.pallas_call(
        _k4_body,
        in_specs=[pl.BlockSpec(memory_space=pl.ANY)
                  if i in (2, 3) else pl.BlockSpec()
                  for i in range(5 + len(ws))],
        out_shape=jax.ShapeDtypeStruct((1, 2), jnp.float32),
        scratch_shapes=[
            pltpu.VMEM((1, FH), jnp.float32),
            pltpu.VMEM((1, FH), jnp.float32),
            pltpu.VMEM((1, F), jnp.float32),
            pltpu.SemaphoreType.DMA,
        ],
    )(scorep, bvec, aggp, x, edges, *ws)


def kernel(x, edge_index, batch, Wr1, Wo1, b1, Wpr1, Wpo1, bp1,
           Wr2, Wo2, b2, Wpr2, Wpo2, bp2, Wr3, Wo3, b3, Wpr3, Wpo3, bp3,
           Wm, bm):
    src = edge_index[0]
    dst = edge_index[1]
    spad = jnp.zeros((EP - E,), jnp.int32)
    dpad = jnp.full((EP - E,), N, jnp.int32)
    srcp = jnp.concatenate([src.astype(jnp.int32), spad]).reshape(
        NW * PPW, RPP, 128)
    dstp = jnp.concatenate([dst.astype(jnp.int32), dpad]).reshape(
        NW * PPW, RPP, 128)
    zeros2d = jnp.zeros((N, FH), jnp.float32)

    aggp = _k1(x, srcp, dstp, zeros2d)

    a3, b3v = _k2(aggp, x,
                  Wr1, Wo1, b1.reshape(1, H),
                  Wpr1.reshape(1, H), Wpo1.reshape(1, H),
                  bp1.reshape(1, 1))
    a1 = a3.reshape(N)
    bvec = b3v.reshape(1, N)

    scorep = _k3(a1, srcp, dstp)

    edges = edge_index.astype(jnp.int32).reshape(2, E // 128, 128)
    wmt = (Wm[:H] + Wm[H:]).T
    return _k4(scorep, bvec, aggp, x, edges,
               Wr1, Wo1, b1.reshape(1, H),
               Wr2, Wo2, b2.reshape(1, H), Wpr2.reshape(1, H),
               Wpo2.reshape(1, H), bp2.reshape(1, 1),
               Wr3, Wo3, b3.reshape(1, H), Wpr3.reshape(1, H),
               Wpo3.reshape(1, H), bp3.reshape(1, 1),
               wmt, bm.reshape(1, 2))


# speedup vs baseline: 3.0020x; 1.0152x over previous
import functools

import jax
import jax.numpy as jnp
from jax import lax
from jax.experimental import pallas as pl
from jax.experimental.pallas import tpu as pltpu
from jax.experimental.pallas import tpu_sc as plsc

N = 10000
E = 320000
F = 128
H = 256
NP = 10240
NC, NS = 2, 16
NW = NC * NS
RPP = 40
PPW = 2
EP = NW * PPW * RPP * 128
ROWS_PER_TILE = NP // NS


FH = F // 2
TPP = 4


def _k1_body(x, srcp, dstp, out, sidx, didx, rows,
             x_sh, agg_sh, gsem, ssem):
    cid = lax.axis_index("c")
    sid = lax.axis_index("s")
    r_base = sid * ROWS_PER_TILE
    @pl.when(sid < NS - 1)
    def _():
        pltpu.sync_copy(x.at[pl.ds(r_base, ROWS_PER_TILE),
                             pl.ds(cid * FH, FH)],
                        x_sh.at[pl.ds(r_base, ROWS_PER_TILE)])

    @pl.when(sid == NS - 1)
    def _():
        last = N - (NS - 1) * ROWS_PER_TILE
        pltpu.sync_copy(x.at[pl.ds(r_base, last), pl.ds(cid * FH, FH)],
                        x_sh.at[pl.ds(r_base, last)])

    def zrow(t, carry):
        rows[0, t // 4, pl.ds((t % 4) * 16, 16)] = jnp.zeros((16,),
                                                             jnp.float32)
        return carry

    lax.fori_loop(0, 128 * FH // 16, zrow, 0)
    for t in range(ROWS_PER_TILE // 128):
        pltpu.sync_copy(rows.at[0],
                        agg_sh.at[pl.ds(r_base + t * 128, 128)])
    plsc.subcore_barrier()

    NB = 4
    for p in range(TPP):
        pltpu.sync_copy(srcp.at[sid * TPP + p], sidx)
        pltpu.sync_copy(dstp.at[sid * TPP + p], didx)
        for b in range(2):
            pltpu.async_copy(x_sh.at[sidx.at[b]], rows.at[b], gsem.at[b])

        def body(t, carry):
            for u in range(NB):
                j = t * NB + u
                b = j % NB
                pltpu.make_async_copy(x_sh.at[sidx.at[j]], rows.at[b],
                                      gsem.at[b]).wait()
                pltpu.async_copy(rows.at[b], agg_sh.at[didx.at[j]],
                                 ssem.at[b], add=True)
                b2 = (j + 2) % NB

                @pl.when(j >= 2)
                def _():
                    pltpu.make_async_copy(rows.at[b2],
                                          agg_sh.at[didx.at[j - 2]],
                                          ssem.at[b2]).wait()

                @pl.when(j + 2 < RPP)
                def _():
                    pltpu.async_copy(x_sh.at[sidx.at[j + 2]], rows.at[b2],
                                     gsem.at[b2])
            return carry

        lax.fori_loop(0, RPP // NB, body, 0)
        for j in (RPP - 2, RPP - 1):
            pltpu.make_async_copy(rows.at[j % NB], agg_sh.at[didx.at[j]],
                                  ssem.at[j % NB]).wait()
    plsc.subcore_barrier()

    def body2(t, carry):
        r0 = r_base + t * 128
        pltpu.sync_copy(agg_sh.at[pl.ds(r0, 128)], rows.at[0])
        pltpu.sync_copy(rows.at[0], out.at[cid].at[pl.ds(r0, 128)])
        return carry

    lax.fori_loop(0, ROWS_PER_TILE // 128, body2, 0)


_k1 = functools.partial(
    pl.kernel,
    out_type=jax.ShapeDtypeStruct((NC, NP, FH), jnp.float32),
    mesh=plsc.VectorSubcoreMesh(core_axis_name="c", subcore_axis_name="s",
                                num_cores=NC, num_subcores=NS),
    scratch_types=[
        pltpu.VMEM((RPP, 128), jnp.int32),
        pltpu.VMEM((RPP, 128), jnp.int32),
        pltpu.VMEM((4, 128, FH), jnp.float32),
        pltpu.VMEM_SHARED((NP, FH), jnp.float32),
        pltpu.VMEM_SHARED((NP, FH), jnp.float32),
        pltpu.SemaphoreType.DMA((4,)),
        pltpu.SemaphoreType.DMA((4,)),
    ],
    compiler_params=pltpu.CompilerParams(use_tc_tiling_on_sc=False),
)(_k1_body)


def _k2_body(agg0, agg1, xb, wr, wo, b1r, wpr, wpo, bp1s, a_out, b_out):
    aggb = jnp.concatenate([agg0[0], agg1[0]], axis=1)
    h = jnp.dot(aggb, wr[...], preferred_element_type=jnp.float32)
    h += jnp.dot(xb[...], wo[...], preferred_element_type=jnp.float32)
    h = jnp.maximum(h + b1r[...], 0.0)
    a_out[...] = jnp.sum(h * wpr[...], axis=1).reshape(1, 1, -1)
    b_out[...] = (jnp.sum(h * wpo[...], axis=1) + bp1s[0, 0]).reshape(1, 1, -1)


def _k2(aggp, x, Wr1, Wo1, b1r, wpr1, wpo1, bp1s):
    R = 1000
    G = N // R
    return pl.pallas_call(
        _k2_body,
        grid=(G,),
        in_specs=[
            pl.BlockSpec((1, R, FH), lambda i: (0, i, 0)),
            pl.BlockSpec((1, R, FH), lambda i: (1, i, 0)),
            pl.BlockSpec((R, F), lambda i: (i, 0)),
            pl.BlockSpec((F, H), lambda i: (0, 0)),
            pl.BlockSpec((F, H), lambda i: (0, 0)),
            pl.BlockSpec((1, H), lambda i: (0, 0)),
            pl.BlockSpec((1, H), lambda i: (0, 0)),
            pl.BlockSpec((1, H), lambda i: (0, 0)),
            pl.BlockSpec((1, 1), lambda i: (0, 0)),
        ],
        out_specs=[
            pl.BlockSpec((1, 1, R), lambda i: (i, 0, 0)),
            pl.BlockSpec((1, 1, R), lambda i: (i, 0, 0)),
        ],
        out_shape=[
            jax.ShapeDtypeStruct((G, 1, R), jnp.float32),
            jax.ShapeDtypeStruct((G, 1, R), jnp.float32),
        ],
    )(aggp, aggp, x, Wr1, Wo1, b1r, wpr1, wpo1, bp1s)


def _k3_body(a_hbm, srcp, dstp, out, sidx, didx, vals, zb, score_sh, a_sh,
             sem):
    cid = lax.axis_index("c")
    sid = lax.axis_index("s")
    w = cid * NS + sid

    def zbody(i, carry):
        zb[pl.ds(i * 16, 16)] = jnp.zeros((16,), jnp.float32)
        return carry

    lax.fori_loop(0, ROWS_PER_TILE // 16, zbody, 0)
    pltpu.sync_copy(zb.at[pl.ds(0, ROWS_PER_TILE)],
                    score_sh.at[pl.ds(sid * ROWS_PER_TILE, ROWS_PER_TILE)])

    @pl.when(sid < NS - 1)
    def _():
        pltpu.sync_copy(a_hbm.at[pl.ds(sid * ROWS_PER_TILE, ROWS_PER_TILE)],
                        a_sh.at[pl.ds(sid * ROWS_PER_TILE, ROWS_PER_TILE)])

    @pl.when(sid == NS - 1)
    def _():
        last = N - (NS - 1) * ROWS_PER_TILE
        pltpu.sync_copy(a_hbm.at[pl.ds(sid * ROWS_PER_TILE, last)],
                        a_sh.at[pl.ds(sid * ROWS_PER_TILE, last)])

    pltpu.sync_copy(srcp.at[pl.ds(w * PPW, PPW)], sidx)
    pltpu.sync_copy(dstp.at[pl.ds(w * PPW, PPW)], didx)
    plsc.subcore_barrier()

    for p in range(PPW):
        def gfire(j, carry):
            pltpu.async_copy(a_sh.at[sidx.at[p].at[j]], vals.at[p].at[j],
                             sem.at[0])
            return carry
        lax.fori_loop(0, RPP, gfire, 0)
    for p in range(PPW):
        def gdrain(j, carry):
            pltpu.make_async_copy(a_sh.at[sidx.at[p].at[j]],
                                  vals.at[p].at[j], sem.at[0]).wait()
            return carry
        lax.fori_loop(0, RPP, gdrain, 0)

    for p in range(PPW):
        def sfire(j, carry):
            pltpu.async_copy(vals.at[p].at[j], score_sh.at[didx.at[p].at[j]],
                             sem.at[1], add=True)
            return carry
        lax.fori_loop(0, RPP, sfire, 0)
    for p in range(PPW):
        def sdrain(j, carry):
            pltpu.make_async_copy(vals.at[p].at[j],
                                  score_sh.at[didx.at[p].at[j]],
                                  sem.at[1]).wait()
            return carry
        lax.fori_loop(0, RPP, sdrain, 0)
    plsc.subcore_barrier()

    @pl.when(sid == 0)
    def _out():
        pltpu.sync_copy(score_sh, zb)
        pltpu.sync_copy(zb, out.at[cid])


_k3 = functools.partial(
    pl.kernel,
    out_type=jax.ShapeDtypeStruct((NC, NP), jnp.float32),
    mesh=plsc.VectorSubcoreMesh(core_axis_name="c", subcore_axis_name="s",
                                num_cores=NC, num_subcores=NS),
    scratch_types=[
        pltpu.VMEM((PPW, RPP, 128), jnp.int32),
        pltpu.VMEM((PPW, RPP, 128), jnp.int32),
        pltpu.VMEM((PPW, RPP, 128), jnp.float32),
        pltpu.VMEM((NP,), jnp.float32),
        pltpu.VMEM_SHARED((NP,), jnp.float32),
        pltpu.VMEM_SHARED((NP,), jnp.float32),
        pltpu.SemaphoreType.DMA((2,)),
    ],
    compiler_params=pltpu.CompilerParams(use_tc_tiling_on_sc=False),
)(_k3_body)


def _k4_body(scorep, bvec, aggp, x, edges,
             wr1, wo1, b1r,
             wr2, wo2, b2r, wpr2, wpo2, bp2s,
             wr3, wo3, b3r, wpr3, wpo3, bp3s,
             wmt, bmr, out, arow0, arow1, xrow, sem):
    s = scorep[0:1, :N] + scorep[1:2, :N] + bvec[...]
    iota = lax.broadcasted_iota(jnp.int32, (1, N), 1)
    m = jnp.max(s)
    idx = jnp.min(jnp.where(s == m, iota, N))
    c0 = pltpu.make_async_copy(aggp.at[0].at[pl.ds(idx, 1)], arow0, sem)
    c0.start()
    c1 = pltpu.make_async_copy(aggp.at[1].at[pl.ds(idx, 1)], arow1, sem)
    c1.start()
    c2 = pltpu.make_async_copy(x.at[pl.ds(idx, 1)], xrow, sem)
    c2.start()
    c0.wait()
    c1.wait()
    c2.wait()
    agg_row = jnp.concatenate([arow0[...], arow1[...]], axis=1)
    hrow = jnp.dot(agg_row, wr1[...], preferred_element_type=jnp.float32)
    hrow += jnp.dot(xrow[...], wo1[...], preferred_element_type=jnp.float32)
    hrow = jnp.maximum(hrow + b1r[...], 0.0)
    xn = hrow * jnp.tanh(m)
    e0 = edges[0]
    e1 = edges[1]
    cf = jnp.sum(jnp.where((e0 == idx) & (e1 == idx), 1.0, 0.0))

    def gconv(v, wr, wo, br):
        y = cf * jnp.dot(v, wr[...], preferred_element_type=jnp.float32)
        y += jnp.dot(v, wo[...], preferred_element_type=jnp.float32)
        return jnp.maximum(y + br[...], 0.0)

    g2 = gconv(xn, wr2, wo2, b2r)
    s2 = cf * jnp.sum(g2 * wpr2[...]) + jnp.sum(g2 * wpo2[...]) + bp2s[0, 0]
    xn2 = g2 * jnp.tanh(s2)
    g3 = gconv(xn2, wr3, wo3, b3r)
    s3 = cf * jnp.sum(g3 * wpr3[...]) + jnp.sum(g3 * wpo3[...]) + bp3s[0, 0]
    xn3 = g3 * jnp.tanh(s3)
    t = xn + xn2 + xn3
    o0 = jnp.sum(t * wmt[0:1, :]) + bmr[0, 0]
    o1 = jnp.sum(t * wmt[1:2, :]) + bmr[0, 1]
    out[...] = jnp.concatenate([o0.reshape(1, 1), o1.reshape(1, 1)], axis=1)


def _k4(scorep, bvec, aggp, x, edges, *ws):
    return pl.pallas_call(
        _k4_body,
        in_specs=[pl.BlockSpec(memory_space=pl.ANY)
                  if i in (2, 3) else pl.BlockSpec()
                  for i in range(5 + len(ws))],
        out_shape=jax.ShapeDtypeStruct((1, 2), jnp.float32),
        scratch_shapes=[
            pltpu.VMEM((1, FH), jnp.float32),
            pltpu.VMEM((1, FH), jnp.float32),
            pltpu.VMEM((1, F), jnp.float32),
            pltpu.SemaphoreType.DMA,
        ],
    )(scorep, bvec, aggp, x, edges, *ws)


def kernel(x, edge_index, batch, Wr1, Wo1, b1, Wpr1, Wpo1, bp1,
           Wr2, Wo2, b2, Wpr2, Wpo2, bp2, Wr3, Wo3, b3, Wpr3, Wpo3, bp3,
           Wm, bm):
    src = edge_index[0]
    dst = edge_index[1]
    spad = jnp.zeros((EP - E,), jnp.int32)
    dpad = jnp.full((EP - E,), N, jnp.int32)
    srcp = jnp.concatenate([src.astype(jnp.int32), spad]).reshape(
        NW * PPW, RPP, 128)
    dstp = jnp.concatenate([dst.astype(jnp.int32), dpad]).reshape(
        NW * PPW, RPP, 128)
    aggp = _k1(x, srcp, dstp)

    a3, b3v = _k2(aggp, x,
                  Wr1, Wo1, b1.reshape(1, H),
                  Wpr1.reshape(1, H), Wpo1.reshape(1, H),
                  bp1.reshape(1, 1))
    a1 = a3.reshape(N)
    bvec = b3v.reshape(1, N)

    scorep = _k3(a1, srcp, dstp)

    edges = edge_index.astype(jnp.int32).reshape(2, E // 128, 128)
    wmt = (Wm[:H] + Wm[H:]).T
    return _k4(scorep, bvec, aggp, x, edges,
               Wr1, Wo1, b1.reshape(1, H),
               Wr2, Wo2, b2.reshape(1, H), Wpr2.reshape(1, H),
               Wpo2.reshape(1, H), bp2.reshape(1, 1),
               Wr3, Wo3, b3.reshape(1, H), Wpr3.reshape(1, H),
               Wpo3.reshape(1, H), bp3.reshape(1, 1),
               wmt, bm.reshape(1, 2))
